# Initial kernel scaffold; baseline (speedup 1.0000x reference)
import functools

import jax
import jax.numpy as jnp
from jax import lax
from jax.experimental import pallas as pl
from jax.experimental.pallas import tpu as pltpu
from jax.experimental.pallas import tpu_sc as plsc

_N = 10000
_D = 128
_FH = 64
_FO = 128
_E = 320000
_G = 64

_NC = 2
_NS = 16
_NW = _NC * _NS
_EPT = _E // _NW
_C = 80
_NCH = _EPT // _C
_RPT = _N // _NS


def _mm_body(x_ref, w_ref, o_ref):
    o_ref[...] = jnp.dot(x_ref[...], w_ref[...],
                         preferred_element_type=jnp.float32)


def _mm(x, w):
    n, _ = x.shape
    _, f = w.shape
    return pl.pallas_call(
        _mm_body,
        out_shape=jax.ShapeDtypeStruct((n, f), jnp.float32),
    )(x, w)


def _bnrelu_body(p_ref, b_ref, g_ref, be_ref, o_ref):
    s = p_ref[0] + p_ref[1] + b_ref[...]
    mu = jnp.mean(s, axis=0, keepdims=True)
    var = jnp.mean((s - mu) ** 2, axis=0, keepdims=True)
    y = (s - mu) * lax.rsqrt(var + 1e-5) * g_ref[...] + be_ref[...]
    o_ref[...] = jnp.maximum(y, 0.0)


def _bnrelu(p, b, g, be):
    return pl.pallas_call(
        _bnrelu_body,
        out_shape=jax.ShapeDtypeStruct((_N, _FH), jnp.float32),
    )(p, b, g, be)


def _final_body(p_ref, w_ref, b_ref, g_ref, be_ref, bv_ref, o_ref):
    s = p_ref[0] + p_ref[1]
    h = jnp.dot(s, w_ref[...], preferred_element_type=jnp.float32) + b_ref[...]
    mu = jnp.mean(h, axis=0, keepdims=True)
    var = jnp.mean((h - mu) ** 2, axis=0, keepdims=True)
    a = jnp.maximum(
        (h - mu) * lax.rsqrt(var + 1e-5) * g_ref[...] + be_ref[...], 0.0)
    gid = lax.broadcasted_iota(jnp.int32, (_G, _N), 0)
    onehot = (gid == bv_ref[...]).astype(jnp.float32)
    sums = jnp.dot(onehot, a, preferred_element_type=jnp.float32)
    counts = jnp.sum(onehot, axis=1, keepdims=True)
    o_ref[...] = sums / jnp.maximum(counts, 1.0)


def _final(p, w, b, g, be, bv):
    return pl.pallas_call(
        _final_body,
        out_shape=jax.ShapeDtypeStruct((_G, _FO), jnp.float32),
    )(p, w, b, g, be, bv)


def _edge_scatter(h, src, dst, ew, zeros):
    mesh = plsc.VectorSubcoreMesh(
        core_axis_name="c", subcore_axis_name="s",
        num_cores=_NC, num_subcores=_NS)

    @functools.partial(
        pl.kernel,
        out_type=jax.ShapeDtypeStruct((_NC, _N, _FH), jnp.float32),
        mesh=mesh,
        scratch_types=[
            pltpu.VMEM_SHARED((_N, _FH), jnp.float32),
            pltpu.VMEM_SHARED((_N, _FH), jnp.float32),
            pltpu.VMEM((_C,), jnp.int32),
            pltpu.VMEM((_C,), jnp.int32),
            pltpu.VMEM((_C,), jnp.float32),
            pltpu.VMEM((_C, _FH), jnp.float32),
            pltpu.SemaphoreType.DMA,
        ],
    )
    def k(h_hbm, src_hbm, dst_hbm, ew_hbm, z_hbm, out_hbm,
          h_sh, acc, idx_s, idx_d, eww, rows, sem):
        cid = lax.axis_index("c")
        sid = lax.axis_index("s")
        wid = cid * _NS + sid
        r0 = sid * _RPT
        pltpu.sync_copy(h_hbm.at[pl.ds(r0, _RPT)], h_sh.at[pl.ds(r0, _RPT)])
        pltpu.sync_copy(z_hbm.at[pl.ds(r0, _RPT)], acc.at[pl.ds(r0, _RPT)])
        plsc.subcore_barrier()
        iota = lax.iota(jnp.int32, 16)

        def chunk_body(ci, carry):
            base = wid * _EPT + ci * _C
            pltpu.sync_copy(src_hbm.at[pl.ds(base, _C)], idx_s)
            pltpu.sync_copy(dst_hbm.at[pl.ds(base, _C)], idx_d)
            pltpu.sync_copy(ew_hbm.at[pl.ds(base, _C)], eww)
            pltpu.async_copy(h_sh.at[idx_s], rows, sem).wait()
            for j in range(_C // 16):
                ew16 = eww[pl.ds(j * 16, 16)]
                ridx = iota + (j * 16)
                for kcol in range(_FH):
                    cidx = jnp.full((16,), kcol, jnp.int32)
                    v = plsc.load_gather(rows, [ridx, cidx])
                    plsc.store_scatter(rows, [ridx, cidx], v * ew16)
            pltpu.sync_copy(rows, acc.at[idx_d], add=True)
            return carry

        lax.fori_loop(0, _NCH, chunk_body, 0)
        plsc.subcore_barrier()
        pltpu.sync_copy(acc.at[pl.ds(r0, _RPT)],
                        out_hbm.at[cid, pl.ds(r0, _RPT)])

    return k(h, src, dst, ew, zeros)


def kernel(x, edge_index, edge_weight, batch_vec,
           W1, b1, g1, be1, W2, b2, g2, be2):
    src = edge_index[0].astype(jnp.int32)
    dst = edge_index[1].astype(jnp.int32)
    ew = edge_weight.astype(jnp.float32)
    bv = batch_vec.astype(jnp.int32).reshape(1, _N)
    z = jnp.zeros((_N, _FH), jnp.float32)

    h1 = _mm(x, W1)
    p1 = _edge_scatter(h1, src, dst, ew, z)
    a1 = _bnrelu(p1, b1.reshape(1, _FH), g1.reshape(1, _FH),
                 be1.reshape(1, _FH))
    p2 = _edge_scatter(a1, src, dst, ew, z)
    out = _final(p2, W2, b2.reshape(1, _FO), g2.reshape(1, _FO),
                 be2.reshape(1, _FO), bv)
    return out


# speedup vs baseline: 1.7471x; 1.7471x over previous
import functools

import jax
import jax.numpy as jnp
from jax import lax
from jax.experimental import pallas as pl
from jax.experimental.pallas import tpu as pltpu
from jax.experimental.pallas import tpu_sc as plsc

_N = 10000
_D = 128
_FH = 64
_FO = 128
_E = 320000
_G = 64

_NC = 2
_NS = 16
_NW = _NC * _NS
_EPT = _E // _NW
_C = 80
_NCH = _EPT // _C
_SRB = 624
_REM = _N - _NS * _SRB


def _mm_body(x_ref, w_ref, o_ref):
    o_ref[...] = jnp.dot(x_ref[...], w_ref[...],
                         preferred_element_type=jnp.float32)


def _mm(x, w):
    n, _ = x.shape
    _, f = w.shape
    return pl.pallas_call(
        _mm_body,
        out_shape=jax.ShapeDtypeStruct((n, f), jnp.float32),
    )(x, w)


def _bnrelu_body(p_ref, b_ref, g_ref, be_ref, o_ref):
    s = p_ref[0] + p_ref[1] + b_ref[...]
    mu = jnp.mean(s, axis=0, keepdims=True)
    var = jnp.mean((s - mu) ** 2, axis=0, keepdims=True)
    y = (s - mu) * lax.rsqrt(var + 1e-5) * g_ref[...] + be_ref[...]
    o_ref[...] = jnp.maximum(y, 0.0)


def _bnrelu(p, b, g, be):
    return pl.pallas_call(
        _bnrelu_body,
        out_shape=jax.ShapeDtypeStruct((_N, _FH), jnp.float32),
    )(p, b, g, be)


def _final_body(p_ref, w_ref, b_ref, g_ref, be_ref, bv_ref, o_ref):
    s = p_ref[0] + p_ref[1]
    h = jnp.dot(s, w_ref[...], preferred_element_type=jnp.float32) + b_ref[...]
    mu = jnp.mean(h, axis=0, keepdims=True)
    var = jnp.mean((h - mu) ** 2, axis=0, keepdims=True)
    a = jnp.maximum(
        (h - mu) * lax.rsqrt(var + 1e-5) * g_ref[...] + be_ref[...], 0.0)
    gid = lax.broadcasted_iota(jnp.int32, (_G, _N), 0)
    onehot = (gid == bv_ref[...]).astype(jnp.float32)
    sums = jnp.dot(onehot, a, preferred_element_type=jnp.float32)
    counts = jnp.sum(onehot, axis=1, keepdims=True)
    o_ref[...] = sums / jnp.maximum(counts, 1.0)


def _final(p, w, b, g, be, bv):
    return pl.pallas_call(
        _final_body,
        out_shape=jax.ShapeDtypeStruct((_G, _FO), jnp.float32),
    )(p, w, b, g, be, bv)


def _edge_scatter(h, src, dst, ew, zeros):
    mesh = plsc.VectorSubcoreMesh(
        core_axis_name="c", subcore_axis_name="s",
        num_cores=_NC, num_subcores=_NS)

    @functools.partial(
        pl.kernel,
        out_type=jax.ShapeDtypeStruct((_NC, _N, _FH), jnp.float32),
        mesh=mesh,
        compiler_params=pltpu.CompilerParams(
            needs_layout_passes=False, use_tc_tiling_on_sc=False),
        scratch_types=[
            pltpu.VMEM_SHARED((_N, _FH), jnp.float32),
            pltpu.VMEM_SHARED((_N, _FH), jnp.float32),
            pltpu.VMEM((_C,), jnp.int32),
            pltpu.VMEM((_C,), jnp.int32),
            pltpu.VMEM((_C,), jnp.float32),
            pltpu.VMEM((_C, _FH), jnp.float32),
            pltpu.VMEM((_C, _FH), jnp.float32),
            pltpu.SemaphoreType.DMA,
        ],
    )
    def k(h_hbm, src_hbm, dst_hbm, ew_hbm, z_hbm, out_hbm,
          h_sh, acc, idx_s, idx_d, eww, rows, scaled, sem):
        cid = lax.axis_index("c")
        sid = lax.axis_index("s")
        wid = cid * _NS + sid
        r0 = sid * _SRB
        pltpu.sync_copy(h_hbm.at[pl.ds(r0, _SRB)], h_sh.at[pl.ds(r0, _SRB)])
        pltpu.sync_copy(z_hbm.at[pl.ds(r0, _SRB)], acc.at[pl.ds(r0, _SRB)])

        @pl.when(sid == _NS - 1)
        def _stage_rem():
            rr = _NS * _SRB
            pltpu.sync_copy(h_hbm.at[pl.ds(rr, _REM)],
                            h_sh.at[pl.ds(rr, _REM)])
            pltpu.sync_copy(z_hbm.at[pl.ds(rr, _REM)],
                            acc.at[pl.ds(rr, _REM)])

        plsc.subcore_barrier()
        iota = lax.iota(jnp.int32, 16)

        def chunk_body(ci, carry):
            base = wid * _EPT + ci * _C
            pltpu.sync_copy(src_hbm.at[pl.ds(base, _C)], idx_s)
            pltpu.sync_copy(dst_hbm.at[pl.ds(base, _C)], idx_d)
            pltpu.sync_copy(ew_hbm.at[pl.ds(base, _C)], eww)
            pltpu.async_copy(h_sh.at[idx_s], rows, sem).wait()
            for j in range(_C // 16):
                ew16 = eww[pl.ds(j * 16, 16)]
                ridx = iota + (j * 16)
                for kcol in range(_FH):
                    cidx = jnp.full((16,), kcol, jnp.int32)
                    v = plsc.load_gather(rows, [ridx, cidx])
                    plsc.store_scatter(scaled, [ridx, cidx], v * ew16)
            pltpu.sync_copy(scaled, acc.at[idx_d], add=True)
            return carry

        lax.fori_loop(0, _NCH, chunk_body, 0)
        plsc.subcore_barrier()
        pltpu.sync_copy(acc.at[pl.ds(r0, _SRB)],
                        out_hbm.at[cid, pl.ds(r0, _SRB)])

        @pl.when(sid == _NS - 1)
        def _write_rem():
            rr = _NS * _SRB
            pltpu.sync_copy(acc.at[pl.ds(rr, _REM)],
                            out_hbm.at[cid, pl.ds(rr, _REM)])

    return k(h, src, dst, ew, zeros)


def kernel(x, edge_index, edge_weight, batch_vec,
           W1, b1, g1, be1, W2, b2, g2, be2):
    src = edge_index[0].astype(jnp.int32)
    dst = edge_index[1].astype(jnp.int32)
    ew = edge_weight.astype(jnp.float32)
    bv = batch_vec.astype(jnp.int32).reshape(1, _N)
    z = jnp.zeros((_N, _FH), jnp.float32)

    h1 = _mm(x, W1)
    p1 = _edge_scatter(h1, src, dst, ew, z)
    a1 = _bnrelu(p1, b1.reshape(1, _FH), g1.reshape(1, _FH),
                 be1.reshape(1, _FH))
    p2 = _edge_scatter(a1, src, dst, ew, z)
    out = _final(p2, W2, b2.reshape(1, _FO), g2.reshape(1, _FO),
                 be2.reshape(1, _FO), bv)
    return out


# speedup vs baseline: 11.8464x; 6.7805x over previous
import functools

import jax
import jax.numpy as jnp
from jax import lax
from jax.experimental import pallas as pl
from jax.experimental.pallas import tpu as pltpu
from jax.experimental.pallas import tpu_sc as plsc

_N = 10000
_D = 128
_FH = 64
_FO = 128
_E = 320000
_G = 64

_NC = 2
_NS = 16
_NW = _NC * _NS
_EPT = _E // _NW
_C = 400
_NCH = _EPT // _C
_SRB = 624
_REM = _N - _NS * _SRB


def _mm_body(x_ref, w_ref, o_ref):
    o_ref[...] = jnp.dot(x_ref[...], w_ref[...],
                         preferred_element_type=jnp.float32)


def _mm(x, w):
    n, _ = x.shape
    _, f = w.shape
    return pl.pallas_call(
        _mm_body,
        out_shape=jax.ShapeDtypeStruct((n, f), jnp.float32),
    )(x, w)


def _bnrelu_body(p_ref, b_ref, g_ref, be_ref, o_ref):
    s = p_ref[0] + p_ref[1] + b_ref[...]
    mu = jnp.mean(s, axis=0, keepdims=True)
    var = jnp.mean((s - mu) ** 2, axis=0, keepdims=True)
    y = (s - mu) * lax.rsqrt(var + 1e-5) * g_ref[...] + be_ref[...]
    o_ref[...] = jnp.maximum(y, 0.0)


def _bnrelu(p, b, g, be):
    return pl.pallas_call(
        _bnrelu_body,
        out_shape=jax.ShapeDtypeStruct((_N, _FH), jnp.float32),
    )(p, b, g, be)


def _final_body(p_ref, w_ref, b_ref, g_ref, be_ref, bv_ref, o_ref):
    s = p_ref[0] + p_ref[1]
    h = jnp.dot(s, w_ref[...], preferred_element_type=jnp.float32) + b_ref[...]
    mu = jnp.mean(h, axis=0, keepdims=True)
    var = jnp.mean((h - mu) ** 2, axis=0, keepdims=True)
    a = jnp.maximum(
        (h - mu) * lax.rsqrt(var + 1e-5) * g_ref[...] + be_ref[...], 0.0)
    gid = lax.broadcasted_iota(jnp.int32, (_G, _N), 0)
    onehot = (gid == bv_ref[...]).astype(jnp.float32)
    sums = jnp.dot(onehot, a, preferred_element_type=jnp.float32)
    counts = jnp.sum(onehot, axis=1, keepdims=True)
    o_ref[...] = sums / jnp.maximum(counts, 1.0)


def _final(p, w, b, g, be, bv):
    return pl.pallas_call(
        _final_body,
        out_shape=jax.ShapeDtypeStruct((_G, _FO), jnp.float32),
    )(p, w, b, g, be, bv)


def _edge_scatter(h, src, dst, ew, zeros):
    mesh = plsc.VectorSubcoreMesh(
        core_axis_name="c", subcore_axis_name="s",
        num_cores=_NC, num_subcores=_NS)

    @functools.partial(
        pl.kernel,
        out_type=jax.ShapeDtypeStruct((_NC, _N, _FH), jnp.float32),
        mesh=mesh,
        compiler_params=pltpu.CompilerParams(
            needs_layout_passes=False, use_tc_tiling_on_sc=False),
        scratch_types=[
            pltpu.VMEM_SHARED((_N, _FH), jnp.float32),
            pltpu.VMEM((2, _C), jnp.int32),
            pltpu.VMEM((2, _C), jnp.int32),
            pltpu.VMEM((2, _C), jnp.int32),
            pltpu.VMEM((2, _C), jnp.float32),
            pltpu.VMEM((2, _C, _FH), jnp.float32),
            pltpu.SemaphoreType.DMA,
            pltpu.SemaphoreType.DMA,
            pltpu.SemaphoreType.DMA,
        ],
    )
    def k(h_hbm, src_hbm, dst_hbm, ew_hbm, z_hbm, out_hbm,
          acc, idx_s, idx_d, idx_d2, eww, rows, semI, semG, semS):
        cid = lax.axis_index("c")
        sid = lax.axis_index("s")
        wid = cid * _NS + sid
        r0 = sid * _SRB

        def start_in(ci, b):
            base = wid * _EPT + ci * _C
            pltpu.async_copy(src_hbm.at[pl.ds(base, _C)], idx_s.at[b], semI)
            pltpu.async_copy(dst_hbm.at[pl.ds(base, _C)], idx_d.at[b], semI)
            pltpu.async_copy(ew_hbm.at[pl.ds(base, _C)], eww.at[b], semI)

        def wait_in(b):
            base = wid * _EPT
            pltpu.make_async_copy(
                src_hbm.at[pl.ds(base, _C)], idx_s.at[b], semI).wait()
            pltpu.make_async_copy(
                dst_hbm.at[pl.ds(base, _C)], idx_d.at[b], semI).wait()
            pltpu.make_async_copy(
                ew_hbm.at[pl.ds(base, _C)], eww.at[b], semI).wait()

        def start_g(b):
            pltpu.async_copy(h_hbm.at[idx_s.at[b]], rows.at[b], semG)

        def wait_g(b):
            pltpu.make_async_copy(
                h_hbm.at[idx_s.at[b]], rows.at[b], semG).wait()

        def start_s(b):
            pltpu.async_copy(rows.at[b], acc.at[idx_d2.at[b]], semS, add=True)

        def wait_s(b):
            pltpu.make_async_copy(
                rows.at[b], acc.at[idx_d2.at[b]], semS).wait()

        start_in(0, 0)
        start_in(1, 1)
        pltpu.sync_copy(z_hbm.at[pl.ds(r0, _SRB)], acc.at[pl.ds(r0, _SRB)])

        @pl.when(sid == _NS - 1)
        def _stage_rem():
            rr = _NS * _SRB
            pltpu.sync_copy(z_hbm.at[pl.ds(rr, _REM)],
                            acc.at[pl.ds(rr, _REM)])

        plsc.subcore_barrier()
        wait_in(0)
        start_g(0)

        def chunk_body(ci, carry):
            b = ci & 1
            nb = 1 - b
            wait_g(b)

            @pl.when(ci >= 1)
            def _():
                wait_s(nb)

            @pl.when(ci + 1 < _NCH)
            def _():
                wait_in(nb)
                start_g(nb)

            for j in range(_C // 16):
                ew16 = eww[b, pl.ds(j * 16, 16)]
                idx_d2[b, pl.ds(j * 16, 16)] = idx_d[b, pl.ds(j * 16, 16)]
                for u in range(16):
                    e = j * 16 + u
                    w = ew16[u]
                    vals = [rows[b, e, pl.ds(k0, 16)]
                            for k0 in range(0, _FH, 16)]
                    for k0, v in zip(range(0, _FH, 16), vals):
                        rows[b, e, pl.ds(k0, 16)] = v * w
            start_s(b)

            @pl.when(ci + 2 < _NCH)
            def _():
                start_in(ci + 2, b)

            return carry

        lax.fori_loop(0, _NCH, chunk_body, 0)
        wait_s((_NCH - 1) & 1)
        plsc.subcore_barrier()
        pltpu.sync_copy(acc.at[pl.ds(r0, _SRB)],
                        out_hbm.at[cid, pl.ds(r0, _SRB)])

        @pl.when(sid == _NS - 1)
        def _write_rem():
            rr = _NS * _SRB
            pltpu.sync_copy(acc.at[pl.ds(rr, _REM)],
                            out_hbm.at[cid, pl.ds(rr, _REM)])

    return k(h, src, dst, ew, zeros)


def kernel(x, edge_index, edge_weight, batch_vec,
           W1, b1, g1, be1, W2, b2, g2, be2):
    src = edge_index[0].astype(jnp.int32)
    dst = edge_index[1].astype(jnp.int32)
    ew = edge_weight.astype(jnp.float32)
    bv = batch_vec.astype(jnp.int32).reshape(1, _N)
    z = jnp.zeros((_N, _FH), jnp.float32)

    h1 = _mm(x, W1)
    p1 = _edge_scatter(h1, src, dst, ew, z)
    a1 = _bnrelu(p1, b1.reshape(1, _FH), g1.reshape(1, _FH),
                 be1.reshape(1, _FH))
    p2 = _edge_scatter(a1, src, dst, ew, z)
    out = _final(p2, W2, b2.reshape(1, _FO), g2.reshape(1, _FO),
                 be2.reshape(1, _FO), bv)
    return out
